# baseline (speedup 1.0000x reference)
import math

import jax
import jax.numpy as jnp
from jax.experimental import pallas as pl
from jax.experimental.pallas import tpu as pltpu

_SIZE = 32000
_SMOOTHING = 0.1
_CONF = 1.0 - _SMOOTHING
_EPS = _SMOOTHING / (_SIZE - 2)
_PAD = 0
_C_ROW = (_SIZE - 2) * _EPS * math.log(_EPS) + _CONF * math.log(_CONF)

_BR = 64
_BC = _SIZE


def _loss_tile(t_ref, x_ref, out_ref):
    r = pl.program_id(0)

    @pl.when(r == 0)
    def _init():
        out_ref[0, 0] = 0.0

    x = x_ref[...]
    t = t_ref[0]
    nonpad = t != _PAD
    col = jax.lax.broadcasted_iota(jnp.int32, (_BR, _BC), 1)
    rs = jnp.sum(x, axis=1, keepdims=True)
    g = jnp.sum(jnp.where(col == t, x, 0.0), axis=1, keepdims=True)
    per_row = -_EPS * rs - (_CONF - _EPS) * g + _EPS * x[:, 0:1] + _C_ROW
    out_ref[0, 0] += jnp.sum(jnp.where(nonpad, per_row, 0.0))


def kernel(x, target):
    N, V = x.shape
    assert V == _SIZE and N % _BR == 0
    nr = N // _BR
    t3 = target.astype(jnp.int32).reshape(nr, _BR, 1)
    out = pl.pallas_call(
        _loss_tile,
        grid=(nr,),
        in_specs=[
            pl.BlockSpec((1, _BR, 1), lambda r: (r, 0, 0)),
            pl.BlockSpec((_BR, _BC), lambda r: (r, 0)),
        ],
        out_specs=pl.BlockSpec(
            (1, 1), lambda r: (0, 0), memory_space=pltpu.SMEM
        ),
        out_shape=jax.ShapeDtypeStruct((1, 1), jnp.float32),
    )(t3, x)
    return out[0, 0]


# speedup vs baseline: 1.0371x; 1.0371x over previous
import math

import jax
import jax.numpy as jnp
from jax.experimental import pallas as pl
from jax.experimental.pallas import tpu as pltpu

_SIZE = 32000
_SMOOTHING = 0.1
_CONF = 1.0 - _SMOOTHING
_EPS = _SMOOTHING / (_SIZE - 2)
_PAD = 0
_C_ROW = (_SIZE - 2) * _EPS * math.log(_EPS) + _CONF * math.log(_CONF)

_BR = 256
_BC = _SIZE


def _loss_tile(t_ref, x_ref, out_ref):
    r = pl.program_id(0)

    @pl.when(r == 0)
    def _init():
        out_ref[0, 0] = 0.0

    x = x_ref[...]
    t = t_ref[0]
    nonpad = t != _PAD
    col = jax.lax.broadcasted_iota(jnp.int32, (_BR, _BC), 1)
    rs = jnp.sum(x, axis=1, keepdims=True)
    g = jnp.sum(jnp.where(col == t, x, 0.0), axis=1, keepdims=True)
    per_row = -_EPS * rs - (_CONF - _EPS) * g + _EPS * x[:, 0:1] + _C_ROW
    out_ref[0, 0] += jnp.sum(jnp.where(nonpad, per_row, 0.0))


def kernel(x, target):
    N, V = x.shape
    assert V == _SIZE and N % _BR == 0
    nr = N // _BR
    t3 = target.astype(jnp.int32).reshape(nr, _BR, 1)
    out = pl.pallas_call(
        _loss_tile,
        grid=(nr,),
        in_specs=[
            pl.BlockSpec((1, _BR, 1), lambda r: (r, 0, 0)),
            pl.BlockSpec((_BR, _BC), lambda r: (r, 0)),
        ],
        out_specs=pl.BlockSpec(
            (1, 1), lambda r: (0, 0), memory_space=pltpu.SMEM
        ),
        out_shape=jax.ShapeDtypeStruct((1, 1), jnp.float32),
        compiler_params=pltpu.CompilerParams(
            vmem_limit_bytes=100 * 1024 * 1024
        ),
    )(t3, x)
    return out[0, 0]
